# baseline (speedup 1.0000x reference)
import functools

import jax
import jax.numpy as jnp
import numpy as np
from jax import lax
from jax.experimental import pallas as pl
from jax.experimental.pallas import tpu as pltpu
from jax.experimental.pallas import tpu_sc as plsc

_N = 10000
_E = 320000
_D = 128
_DEG_HIST = np.concatenate([np.zeros(25), np.full(16, 625.0)])
_bins = np.arange(_DEG_HIST.shape[0], dtype=np.float64)
_AVG_LOG = float((np.log(_bins + 1.0) * _DEG_HIST).sum() / _DEG_HIST.sum())

_NPAD = 10240
_NW = 160
_C = 2000
_NCHUNK = _E // _C
_B = 64
_CAP = _C + _B
_BIG = 3.0e38


def _node_body(x_ref, w_ref, o1, o2, o3, o4):
    y = jnp.dot(x_ref[...], w_ref[...], preferred_element_type=jnp.float32)
    o1[...] = y[:, :_D]
    o2[...] = y[:, _D:2 * _D]
    o3[...] = y[:, 2 * _D:3 * _D]
    o4[...] = y[:, 3 * _D:]


def _node_tables(x_pad, wnode):
    blk = 1280
    g = _NPAD // blk
    out = jax.ShapeDtypeStruct((_NPAD, _D), jnp.float32)
    return pl.pallas_call(
        _node_body,
        grid=(g,),
        in_specs=[
            pl.BlockSpec((blk, _D), lambda i: (i, 0)),
            pl.BlockSpec((_D, 4 * _D), lambda i: (0, 0)),
        ],
        out_specs=[pl.BlockSpec((blk, _D), lambda i: (i, 0))] * 4,
        out_shape=[out] * 4,
    )(x_pad, wnode)


def _edge_body(ea_ref, w_ref, b_ref, o1, o2):
    y = (jnp.dot(ea_ref[...], w_ref[...], preferred_element_type=jnp.float32)
         + b_ref[...])
    o1[...] = y[:, :_D]
    o2[...] = y[:, _D:]


def _edge_tables(edge_attr, wea, bias):
    blk = 4000
    g = _E // blk
    ed = edge_attr.shape[1]
    out = jax.ShapeDtypeStruct((_E, _D), jnp.float32)
    return pl.pallas_call(
        _edge_body,
        grid=(g,),
        in_specs=[
            pl.BlockSpec((blk, ed), lambda i: (i, 0)),
            pl.BlockSpec((ed, 2 * _D), lambda i: (0, 0)),
            pl.BlockSpec((2 * _D,), lambda i: (0,)),
        ],
        out_specs=[pl.BlockSpec((blk, _D), lambda i: (i, 0))] * 2,
        out_shape=[out] * 2,
    )(edge_attr, wea, bias)


def _sc_body(dst_h, src_h, me_h, as_h,
             sum_h, sq_h, mn_h, mx_h, cnt_h,
             accS, accQ, accMn, accMx, cntT,
             dstc0, dstc1, srcc0, srcc1, ceid, csrc, cdst,
             gAs0, gAs1, gme0, gme1, eIdx0, eIdx1, sIdx0, sIdx1,
             semD, semS, semM0, semM1, semA0, semA1):
    cid = lax.axis_index("c")
    sid = lax.axis_index("s")
    wid = sid * 2 + cid
    zeros_f = jnp.zeros((16,), jnp.float32)
    zeros_i = jnp.zeros((16,), jnp.int32)
    ones_f = jnp.ones((16,), jnp.float32)
    big_f = jnp.full((16,), _BIG, jnp.float32)
    iota16 = lax.iota(jnp.int32, 16)
    dstc = (dstc0, dstc1)
    srcc = (srcc0, srcc1)
    gAs = (gAs0, gAs1)
    gme = (gme0, gme1)
    eIdx = (eIdx0, eIdx1)
    sIdx = (sIdx0, sIdx1)
    semM = (semM0, semM1)
    semA = (semA0, semA1)

    def zb(i, c):
        s = pl.ds(i * 16, 16)
        ceid[s] = zeros_i
        csrc[s] = zeros_i
        cdst[s] = zeros_i
        return c
    lax.fori_loop(0, _CAP // 16, zb, 0)

    def stage(slot, off):
        for t in range(_B // 16):
            sl = pl.ds(off + t * 16, 16)
            eIdx[slot][pl.ds(t * 16, 16)] = ceid[sl]
            sIdx[slot][pl.ds(t * 16, 16)] = csrc[sl]

    def fire(slot):
        pltpu.async_copy(me_h.at[eIdx[slot]], gme[slot], semM[slot])
        pltpu.async_copy(as_h.at[sIdx[slot]], gAs[slot], semA[slot])

    def drain(slot):
        pltpu.make_async_copy(me_h.at[eIdx[slot]], gme[slot],
                              semM[slot]).wait()
        pltpu.make_async_copy(as_h.at[sIdx[slot]], gAs[slot],
                              semA[slot]).wait()

    def accum_batch(slot, off, n, full):
        ga = gAs[slot]
        gm = gme[slot]

        def vgroup(v, c):
            base = v * 16
            dlocv = cdst[pl.ds(off + base, 16)]
            for l in range(16):
                j = base + l
                dloc = dlocv[l]

                def accum():
                    for q in range(8):
                        s = pl.ds(q * 16, 16)
                        mq = ga[j, s] + gm[j, s]
                        plsc.addupdate(accS.at[dloc, s], mq)
                        plsc.addupdate(accQ.at[dloc, s], mq * mq)
                        accMn[dloc, s] = jnp.minimum(accMn[dloc, s], mq)
                        accMx[dloc, s] = jnp.maximum(accMx[dloc, s], mq)

                if full:
                    accum()
                else:
                    pl.when(j < n)(accum)
            return c
        lax.fori_loop(0, _B // 16, vgroup, 0)

    def wbody(w, wc):
        lo = wid * 320 + w * _NW

        def init_row(i, c):
            for q in range(8):
                s = pl.ds(q * 16, 16)
                accS[i, s] = zeros_f
                accQ[i, s] = zeros_f
                accMn[i, s] = big_f
                accMx[i, s] = -big_f
            return c
        lax.fori_loop(0, _NW, init_row, 0)

        def init_cnt(i, c):
            cntT[pl.ds(i * 16, 16)] = zeros_f
            return c
        lax.fori_loop(0, _NW // 16, init_cnt, 0)

        pltpu.sync_copy(dst_h.at[pl.ds(0, _C)], dstc[0])
        pltpu.sync_copy(src_h.at[pl.ds(0, _C)], srcc[0])

        def half_chunk(ci, q, cursor):
            nxt = jnp.minimum(ci + 1, _NCHUNK - 1) * _C
            cD = pltpu.async_copy(dst_h.at[pl.ds(nxt, _C)], dstc[1 - q],
                                  semD)
            cS = pltpu.async_copy(src_h.at[pl.ds(nxt, _C)], srcc[1 - q],
                                  semS)
            cb = ci * _C
            dc = dstc[q]
            sc = srcc[q]

            def vbody(v, cur):
                s = pl.ds(v * 16, 16)
                dv = dc[s]
                sv = sc[s]
                dloc = dv - lo
                msk = (dloc >= 0) & (dloc < _NW)
                ev = cb + v * 16 + iota16
                plsc.store_compressed(ceid.at[pl.ds(cur, 16)], ev, mask=msk)
                plsc.store_compressed(csrc.at[pl.ds(cur, 16)], sv, mask=msk)
                plsc.store_compressed(cdst.at[pl.ds(cur, 16)], dloc, mask=msk)
                plsc.addupdate_scatter(cntT, [dloc], ones_f, mask=msk)
                return cur + jnp.sum(msk.astype(jnp.int32))
            cursor = lax.fori_loop(0, _C // 16, vbody, cursor)

            nb = cursor >> 6

            def prime():
                stage(0, 0)
                fire(0)
            pl.when(nb > 0)(prime)

            def ring(g, c):
                for b in range(2):
                    bi = g * 2 + b

                    def step():
                        def launch_next():
                            stage(1 - b, (bi + 1) * _B)
                            fire(1 - b)
                        pl.when(bi + 1 < nb)(launch_next)
                        drain(b)
                        accum_batch(b, bi * _B, _B, True)
                    pl.when(bi < nb)(step)
                return c
            lax.fori_loop(0, (nb + 1) >> 1, ring, 0)

            off = nb * _B
            for t in range(_B // 16):
                sl_d = pl.ds(t * 16, 16)
                sl_s = pl.ds(off + t * 16, 16)
                ev = ceid[sl_s]
                sv = csrc[sl_s]
                dv = cdst[sl_s]
                ceid[sl_d] = ev
                csrc[sl_d] = sv
                cdst[sl_d] = dv
            cD.wait()
            cS.wait()
            return cursor - off

        def chunk_pair(g, cursor):
            cursor = half_chunk(g * 2, 0, cursor)
            cursor = half_chunk(g * 2 + 1, 1, cursor)
            return cursor

        cursor = lax.fori_loop(0, _NCHUNK // 2, chunk_pair, jnp.int32(0))
        stage(0, 0)
        fire(0)
        drain(0)
        accum_batch(0, 0, cursor, False)

        pltpu.sync_copy(accS, sum_h.at[pl.ds(lo, _NW)])
        pltpu.sync_copy(accQ, sq_h.at[pl.ds(lo, _NW)])
        pltpu.sync_copy(accMn, mn_h.at[pl.ds(lo, _NW)])
        pltpu.sync_copy(accMx, mx_h.at[pl.ds(lo, _NW)])
        pltpu.sync_copy(cntT, cnt_h.at[pl.ds(lo, _NW)])
        return wc

    lax.fori_loop(0, 2, wbody, 0)


def _sc_stats(dst, src, me, as_):
    mesh = plsc.VectorSubcoreMesh(core_axis_name="c", subcore_axis_name="s",
                                  num_cores=2, num_subcores=16)
    mat = jax.ShapeDtypeStruct((_NPAD, _D), jnp.float32)
    vec = jax.ShapeDtypeStruct((_NPAD,), jnp.float32)
    f = pl.kernel(
        _sc_body,
        out_type=[mat, mat, mat, mat, vec],
        mesh=mesh,
        compiler_params=pltpu.CompilerParams(needs_layout_passes=False),
        scratch_types=[
            pltpu.VMEM((_NW, _D), jnp.float32),
            pltpu.VMEM((_NW, _D), jnp.float32),
            pltpu.VMEM((_NW, _D), jnp.float32),
            pltpu.VMEM((_NW, _D), jnp.float32),
            pltpu.VMEM((_NW,), jnp.float32),
            pltpu.VMEM((_C,), jnp.int32),
            pltpu.VMEM((_C,), jnp.int32),
            pltpu.VMEM((_C,), jnp.int32),
            pltpu.VMEM((_C,), jnp.int32),
            pltpu.VMEM((_CAP,), jnp.int32),
            pltpu.VMEM((_CAP,), jnp.int32),
            pltpu.VMEM((_CAP,), jnp.int32),
            pltpu.VMEM((_B, _D), jnp.float32),
            pltpu.VMEM((_B, _D), jnp.float32),
            pltpu.VMEM((_B, _D), jnp.float32),
            pltpu.VMEM((_B, _D), jnp.float32),
            pltpu.VMEM((_B,), jnp.int32),
            pltpu.VMEM((_B,), jnp.int32),
            pltpu.VMEM((_B,), jnp.int32),
            pltpu.VMEM((_B,), jnp.int32),
            pltpu.SemaphoreType.DMA,
            pltpu.SemaphoreType.DMA,
            pltpu.SemaphoreType.DMA,
            pltpu.SemaphoreType.DMA,
            pltpu.SemaphoreType.DMA,
            pltpu.SemaphoreType.DMA,
        ],
    )
    return f(dst, src, me, as_)


def _post_body(x_ref, adf, sf, qf, nf, xf, cf, adb, sb, qb, nb, xb, cb,
               wpf, bpf, wlf, blf, wpb, bpb, wlb, blb, whl, bhl, o_ref):
    x = x_ref[...]

    def dir_out(ad_ref, s, q, mn, mx, c, wpost, bpost, wlin, blin):
        ad = ad_ref[...]
        cnt = c[...]
        denom = jnp.maximum(cnt, 1.0)
        has = cnt > 0.0
        adv = jnp.where(has, ad, 0.0)
        meanG = s[...] / denom
        mean = meanG + adv
        mean2 = q[...] / denom + 2.0 * adv * meanG + adv * adv
        var = mean2 - mean * mean
        std = jnp.where(var <= 1e-5, 0.0,
                        jnp.sqrt(jnp.maximum(var, 1e-5)))
        mnv = jnp.where(has, mn[...] + ad, 0.0)
        mxv = jnp.where(has, mx[...] + ad, 0.0)
        agg = jnp.concatenate([mean, mnv, mxv, std], axis=-1)
        logd = jnp.log(denom + 1.0)
        h = jnp.concatenate(
            [x, agg, agg * (logd / _AVG_LOG), agg * (_AVG_LOG / logd)],
            axis=-1)
        p = (jnp.dot(h, wpost[...], preferred_element_type=jnp.float32)
             + bpost[...])
        return (jnp.dot(p, wlin[...], preferred_element_type=jnp.float32)
                + blin[...])

    a_in = dir_out(adf, sf, qf, nf, xf, cf, wpf, bpf, wlf, blf)
    a_out = dir_out(adb, sb, qb, nb, xb, cb, wpb, bpb, wlb, blb)
    hcat = jnp.concatenate([x, a_in, a_out], axis=-1)
    o_ref[...] = (jnp.dot(hcat, whl[...], preferred_element_type=jnp.float32)
                  + bhl[...])


def _post(x, stats_f, stats_b, wpf, bpf, wlf, blf, wpb, bpb, wlb, blb,
          whl, bhl):
    blk = 1000
    g = _N // blk
    m_spec = pl.BlockSpec((blk, _D), lambda i: (i, 0))
    c_spec = pl.BlockSpec((blk, 1), lambda i: (i, 0))
    w13 = pl.BlockSpec((13 * _D, _D), lambda i: (0, 0))
    wdd = pl.BlockSpec((_D, _D), lambda i: (0, 0))
    w3 = pl.BlockSpec((3 * _D, _D), lambda i: (0, 0))
    b_spec = pl.BlockSpec((_D,), lambda i: (0,))
    return pl.pallas_call(
        _post_body,
        grid=(g,),
        in_specs=[m_spec,
                  m_spec, m_spec, m_spec, m_spec, m_spec, c_spec,
                  m_spec, m_spec, m_spec, m_spec, m_spec, c_spec,
                  w13, b_spec, wdd, b_spec,
                  w13, b_spec, wdd, b_spec,
                  w3, b_spec],
        out_specs=pl.BlockSpec((blk, _D), lambda i: (i, 0)),
        out_shape=jax.ShapeDtypeStruct((_N, _D), jnp.float32),
    )(x, *stats_f, *stats_b, wpf, bpf, wlf, blf, wpb, bpb, wlb, blb,
      whl, bhl)


def kernel(x, edge_index, edge_attr, We_f, be_f, Wpre_f, bpre_f, Wpost_f,
           bpost_f, Wlin_f, blin_f, We_b, be_b, Wpre_b, bpre_b, Wpost_b,
           bpost_b, Wlin_b, blin_b, Whl, bhl):
    src = edge_index[0]
    dst = edge_index[1]
    x_pad = jnp.pad(x, ((0, _NPAD - _N), (0, 0)))
    wnode = jnp.concatenate(
        [Wpre_f[:_D], Wpre_f[_D:2 * _D], Wpre_b[:_D], Wpre_b[_D:2 * _D]],
        axis=1)
    ad_f, as_f, ad_b, as_b = _node_tables(x_pad, wnode)
    wea = jnp.concatenate(
        [We_f @ Wpre_f[2 * _D:], We_b @ Wpre_b[2 * _D:]], axis=1)
    bias_e = jnp.concatenate(
        [be_f @ Wpre_f[2 * _D:] + bpre_f, be_b @ Wpre_b[2 * _D:] + bpre_b])
    me_f, me_b = _edge_tables(edge_attr, wea, bias_e)

    s_f, q_f, n_f, x_f, c_f = _sc_stats(dst, src, me_f, as_f)
    s_b, q_b, n_b, x_b, c_b = _sc_stats(src, dst, me_b, as_b)

    stats_f = (ad_f[:_N], s_f[:_N], q_f[:_N], n_f[:_N], x_f[:_N],
               c_f[:_N].reshape(_N, 1))
    stats_b = (ad_b[:_N], s_b[:_N], q_b[:_N], n_b[:_N], x_b[:_N],
               c_b[:_N].reshape(_N, 1))
    return _post(x, stats_f, stats_b, Wpost_f, bpost_f, Wlin_f, blin_f,
                 Wpost_b, bpost_b, Wlin_b, blin_b, Whl, bhl)


# speedup vs baseline: 1.0900x; 1.0900x over previous
import functools

import jax
import jax.numpy as jnp
import numpy as np
from jax import lax
from jax.experimental import pallas as pl
from jax.experimental.pallas import tpu as pltpu
from jax.experimental.pallas import tpu_sc as plsc

_N = 10000
_E = 320000
_D = 128
_DEG_HIST = np.concatenate([np.zeros(25), np.full(16, 625.0)])
_bins = np.arange(_DEG_HIST.shape[0], dtype=np.float64)
_AVG_LOG = float((np.log(_bins + 1.0) * _DEG_HIST).sum() / _DEG_HIST.sum())

_NPAD = 10240
_NW = 160
_C = 2000
_NCHUNK = _E // _C
_B = 128
_CAP = _C + _B
_BIG = 3.0e38


def _node_body(x_ref, w_ref, o1, o2, o3, o4):
    y = jnp.dot(x_ref[...], w_ref[...], preferred_element_type=jnp.float32)
    o1[...] = y[:, :_D]
    o2[...] = y[:, _D:2 * _D]
    o3[...] = y[:, 2 * _D:3 * _D]
    o4[...] = y[:, 3 * _D:]


def _node_tables(x_pad, wnode):
    blk = 1280
    g = _NPAD // blk
    out = jax.ShapeDtypeStruct((_NPAD, _D), jnp.float32)
    return pl.pallas_call(
        _node_body,
        grid=(g,),
        in_specs=[
            pl.BlockSpec((blk, _D), lambda i: (i, 0)),
            pl.BlockSpec((_D, 4 * _D), lambda i: (0, 0)),
        ],
        out_specs=[pl.BlockSpec((blk, _D), lambda i: (i, 0))] * 4,
        out_shape=[out] * 4,
    )(x_pad, wnode)


def _edge_body(ea_ref, w_ref, b_ref, o1, o2):
    y = (jnp.dot(ea_ref[...], w_ref[...], preferred_element_type=jnp.float32)
         + b_ref[...])
    o1[...] = y[:, :_D]
    o2[...] = y[:, _D:]


def _edge_tables(edge_attr, wea, bias):
    blk = 4000
    g = _E // blk
    ed = edge_attr.shape[1]
    out = jax.ShapeDtypeStruct((_E, _D), jnp.float32)
    return pl.pallas_call(
        _edge_body,
        grid=(g,),
        in_specs=[
            pl.BlockSpec((blk, ed), lambda i: (i, 0)),
            pl.BlockSpec((ed, 2 * _D), lambda i: (0, 0)),
            pl.BlockSpec((2 * _D,), lambda i: (0,)),
        ],
        out_specs=[pl.BlockSpec((blk, _D), lambda i: (i, 0))] * 2,
        out_shape=[out] * 2,
    )(edge_attr, wea, bias)


def _sc_body(dst_h, src_h, me_h, as_h,
             sum_h, sq_h, mn_h, mx_h, cnt_h,
             accS, accQ, accMn, accMx, cntT,
             dstc0, dstc1, srcc0, srcc1, ceid, csrc, cdst,
             gAs, gme, eIdx, sIdx,
             semD, semS, sem1, sem2):
    cid = lax.axis_index("c")
    sid = lax.axis_index("s")
    wid = sid * 2 + cid
    zeros_f = jnp.zeros((16,), jnp.float32)
    zeros_i = jnp.zeros((16,), jnp.int32)
    ones_f = jnp.ones((16,), jnp.float32)
    big_f = jnp.full((16,), _BIG, jnp.float32)
    iota16 = lax.iota(jnp.int32, 16)
    dstc = (dstc0, dstc1)
    srcc = (srcc0, srcc1)

    def zb(i, c):
        s = pl.ds(i * 16, 16)
        ceid[s] = zeros_i
        csrc[s] = zeros_i
        cdst[s] = zeros_i
        return c
    lax.fori_loop(0, _CAP // 16, zb, 0)

    def process_batch(off, n, full):
        for t in range(_B // 16):
            sl = pl.ds(off + t * 16, 16)
            eIdx[pl.ds(t * 16, 16)] = ceid[sl]
            sIdx[pl.ds(t * 16, 16)] = csrc[sl]
        c1 = pltpu.async_copy(me_h.at[eIdx], gme, sem1)
        c2 = pltpu.async_copy(as_h.at[sIdx], gAs, sem2)
        c1.wait()
        c2.wait()
        ga = gAs
        gm = gme

        def vgroup(v, c):
            base = v * 16
            dlocv = cdst[pl.ds(off + base, 16)]
            for l in range(16):
                j = base + l
                dloc = dlocv[l]

                def accum():
                    for q in range(8):
                        s = pl.ds(q * 16, 16)
                        mq = ga[j, s] + gm[j, s]
                        plsc.addupdate(accS.at[dloc, s], mq)
                        plsc.addupdate(accQ.at[dloc, s], mq * mq)
                        accMn[dloc, s] = jnp.minimum(accMn[dloc, s], mq)
                        accMx[dloc, s] = jnp.maximum(accMx[dloc, s], mq)

                if full:
                    accum()
                else:
                    pl.when(j < n)(accum)
            return c
        lax.fori_loop(0, _B // 16, vgroup, 0)

    def wbody(w, wc):
        lo = wid * 320 + w * _NW

        def init_row(i, c):
            for q in range(8):
                s = pl.ds(q * 16, 16)
                accS[i, s] = zeros_f
                accQ[i, s] = zeros_f
                accMn[i, s] = big_f
                accMx[i, s] = -big_f
            return c
        lax.fori_loop(0, _NW, init_row, 0)

        def init_cnt(i, c):
            cntT[pl.ds(i * 16, 16)] = zeros_f
            return c
        lax.fori_loop(0, _NW // 16, init_cnt, 0)

        pltpu.sync_copy(dst_h.at[pl.ds(0, _C)], dstc[0])
        pltpu.sync_copy(src_h.at[pl.ds(0, _C)], srcc[0])

        def half_chunk(ci, q, cursor):
            nxt = jnp.minimum(ci + 1, _NCHUNK - 1) * _C
            cD = pltpu.async_copy(dst_h.at[pl.ds(nxt, _C)], dstc[1 - q],
                                  semD)
            cS = pltpu.async_copy(src_h.at[pl.ds(nxt, _C)], srcc[1 - q],
                                  semS)
            cb = ci * _C
            dc = dstc[q]
            sc = srcc[q]

            def vbody(v, cur):
                s = pl.ds(v * 16, 16)
                dv = dc[s]
                sv = sc[s]
                dloc = dv - lo
                msk = (dloc >= 0) & (dloc < _NW)
                ev = cb + v * 16 + iota16
                plsc.store_compressed(ceid.at[pl.ds(cur, 16)], ev, mask=msk)
                plsc.store_compressed(csrc.at[pl.ds(cur, 16)], sv, mask=msk)
                plsc.store_compressed(cdst.at[pl.ds(cur, 16)], dloc, mask=msk)
                plsc.addupdate_scatter(cntT, [dloc], ones_f, mask=msk)
                return cur + jnp.sum(msk.astype(jnp.int32))
            cursor = lax.fori_loop(0, _C // 16, vbody, cursor)

            nb = cursor >> 7

            def bbody(b, c):
                process_batch(b * _B, _B, True)
                return c
            lax.fori_loop(0, nb, bbody, 0)

            off = nb * _B
            for t in range(_B // 16):
                sl_d = pl.ds(t * 16, 16)
                sl_s = pl.ds(off + t * 16, 16)
                ev = ceid[sl_s]
                sv = csrc[sl_s]
                dv = cdst[sl_s]
                ceid[sl_d] = ev
                csrc[sl_d] = sv
                cdst[sl_d] = dv
            cD.wait()
            cS.wait()
            return cursor - off

        def chunk_pair(g, cursor):
            cursor = half_chunk(g * 2, 0, cursor)
            cursor = half_chunk(g * 2 + 1, 1, cursor)
            return cursor

        cursor = lax.fori_loop(0, _NCHUNK // 2, chunk_pair, jnp.int32(0))
        process_batch(0, cursor, False)

        pltpu.sync_copy(accS, sum_h.at[pl.ds(lo, _NW)])
        pltpu.sync_copy(accQ, sq_h.at[pl.ds(lo, _NW)])
        pltpu.sync_copy(accMn, mn_h.at[pl.ds(lo, _NW)])
        pltpu.sync_copy(accMx, mx_h.at[pl.ds(lo, _NW)])
        pltpu.sync_copy(cntT, cnt_h.at[pl.ds(lo, _NW)])
        return wc

    lax.fori_loop(0, 2, wbody, 0)


def _sc_stats(dst, src, me, as_):
    mesh = plsc.VectorSubcoreMesh(core_axis_name="c", subcore_axis_name="s",
                                  num_cores=2, num_subcores=16)
    mat = jax.ShapeDtypeStruct((_NPAD, _D), jnp.float32)
    vec = jax.ShapeDtypeStruct((_NPAD,), jnp.float32)
    f = pl.kernel(
        _sc_body,
        out_type=[mat, mat, mat, mat, vec],
        mesh=mesh,
        compiler_params=pltpu.CompilerParams(needs_layout_passes=False),
        scratch_types=[
            pltpu.VMEM((_NW, _D), jnp.float32),
            pltpu.VMEM((_NW, _D), jnp.float32),
            pltpu.VMEM((_NW, _D), jnp.float32),
            pltpu.VMEM((_NW, _D), jnp.float32),
            pltpu.VMEM((_NW,), jnp.float32),
            pltpu.VMEM((_C,), jnp.int32),
            pltpu.VMEM((_C,), jnp.int32),
            pltpu.VMEM((_C,), jnp.int32),
            pltpu.VMEM((_C,), jnp.int32),
            pltpu.VMEM((_CAP,), jnp.int32),
            pltpu.VMEM((_CAP,), jnp.int32),
            pltpu.VMEM((_CAP,), jnp.int32),
            pltpu.VMEM((_B, _D), jnp.float32),
            pltpu.VMEM((_B, _D), jnp.float32),
            pltpu.VMEM((_B,), jnp.int32),
            pltpu.VMEM((_B,), jnp.int32),
            pltpu.SemaphoreType.DMA,
            pltpu.SemaphoreType.DMA,
            pltpu.SemaphoreType.DMA,
            pltpu.SemaphoreType.DMA,
        ],
    )
    return f(dst, src, me, as_)


def _post_body(x_ref, adf, sf, qf, nf, xf, cf, adb, sb, qb, nb, xb, cb,
               wpf, bpf, wlf, blf, wpb, bpb, wlb, blb, whl, bhl, o_ref):
    x = x_ref[...]

    def dir_out(ad_ref, s, q, mn, mx, c, wpost, bpost, wlin, blin):
        ad = ad_ref[...]
        cnt = c[...]
        denom = jnp.maximum(cnt, 1.0)
        has = cnt > 0.0
        adv = jnp.where(has, ad, 0.0)
        meanG = s[...] / denom
        mean = meanG + adv
        mean2 = q[...] / denom + 2.0 * adv * meanG + adv * adv
        var = mean2 - mean * mean
        std = jnp.where(var <= 1e-5, 0.0,
                        jnp.sqrt(jnp.maximum(var, 1e-5)))
        mnv = jnp.where(has, mn[...] + ad, 0.0)
        mxv = jnp.where(has, mx[...] + ad, 0.0)
        agg = jnp.concatenate([mean, mnv, mxv, std], axis=-1)
        logd = jnp.log(denom + 1.0)
        h = jnp.concatenate(
            [x, agg, agg * (logd / _AVG_LOG), agg * (_AVG_LOG / logd)],
            axis=-1)
        p = (jnp.dot(h, wpost[...], preferred_element_type=jnp.float32)
             + bpost[...])
        return (jnp.dot(p, wlin[...], preferred_element_type=jnp.float32)
                + blin[...])

    a_in = dir_out(adf, sf, qf, nf, xf, cf, wpf, bpf, wlf, blf)
    a_out = dir_out(adb, sb, qb, nb, xb, cb, wpb, bpb, wlb, blb)
    hcat = jnp.concatenate([x, a_in, a_out], axis=-1)
    o_ref[...] = (jnp.dot(hcat, whl[...], preferred_element_type=jnp.float32)
                  + bhl[...])


def _post(x, stats_f, stats_b, wpf, bpf, wlf, blf, wpb, bpb, wlb, blb,
          whl, bhl):
    blk = 1000
    g = _N // blk
    m_spec = pl.BlockSpec((blk, _D), lambda i: (i, 0))
    c_spec = pl.BlockSpec((blk, 1), lambda i: (i, 0))
    w13 = pl.BlockSpec((13 * _D, _D), lambda i: (0, 0))
    wdd = pl.BlockSpec((_D, _D), lambda i: (0, 0))
    w3 = pl.BlockSpec((3 * _D, _D), lambda i: (0, 0))
    b_spec = pl.BlockSpec((_D,), lambda i: (0,))
    return pl.pallas_call(
        _post_body,
        grid=(g,),
        in_specs=[m_spec,
                  m_spec, m_spec, m_spec, m_spec, m_spec, c_spec,
                  m_spec, m_spec, m_spec, m_spec, m_spec, c_spec,
                  w13, b_spec, wdd, b_spec,
                  w13, b_spec, wdd, b_spec,
                  w3, b_spec],
        out_specs=pl.BlockSpec((blk, _D), lambda i: (i, 0)),
        out_shape=jax.ShapeDtypeStruct((_N, _D), jnp.float32),
    )(x, *stats_f, *stats_b, wpf, bpf, wlf, blf, wpb, bpb, wlb, blb,
      whl, bhl)


def kernel(x, edge_index, edge_attr, We_f, be_f, Wpre_f, bpre_f, Wpost_f,
           bpost_f, Wlin_f, blin_f, We_b, be_b, Wpre_b, bpre_b, Wpost_b,
           bpost_b, Wlin_b, blin_b, Whl, bhl):
    src = edge_index[0]
    dst = edge_index[1]
    x_pad = jnp.pad(x, ((0, _NPAD - _N), (0, 0)))
    wnode = jnp.concatenate(
        [Wpre_f[:_D], Wpre_f[_D:2 * _D], Wpre_b[:_D], Wpre_b[_D:2 * _D]],
        axis=1)
    ad_f, as_f, ad_b, as_b = _node_tables(x_pad, wnode)
    wea = jnp.concatenate(
        [We_f @ Wpre_f[2 * _D:], We_b @ Wpre_b[2 * _D:]], axis=1)
    bias_e = jnp.concatenate(
        [be_f @ Wpre_f[2 * _D:] + bpre_f, be_b @ Wpre_b[2 * _D:] + bpre_b])
    me_f, me_b = _edge_tables(edge_attr, wea, bias_e)

    s_f, q_f, n_f, x_f, c_f = _sc_stats(dst, src, me_f, as_f)
    s_b, q_b, n_b, x_b, c_b = _sc_stats(src, dst, me_b, as_b)

    stats_f = (ad_f[:_N], s_f[:_N], q_f[:_N], n_f[:_N], x_f[:_N],
               c_f[:_N].reshape(_N, 1))
    stats_b = (ad_b[:_N], s_b[:_N], q_b[:_N], n_b[:_N], x_b[:_N],
               c_b[:_N].reshape(_N, 1))
    return _post(x, stats_f, stats_b, Wpost_f, bpost_f, Wlin_f, blin_f,
                 Wpost_b, bpost_b, Wlin_b, blin_b, Whl, bhl)


# speedup vs baseline: 1.1575x; 1.0620x over previous
import functools

import jax
import jax.numpy as jnp
import numpy as np
from jax import lax
from jax.experimental import pallas as pl
from jax.experimental.pallas import tpu as pltpu
from jax.experimental.pallas import tpu_sc as plsc

_N = 10000
_E = 320000
_D = 128
_DEG_HIST = np.concatenate([np.zeros(25), np.full(16, 625.0)])
_bins = np.arange(_DEG_HIST.shape[0], dtype=np.float64)
_AVG_LOG = float((np.log(_bins + 1.0) * _DEG_HIST).sum() / _DEG_HIST.sum())

_NPAD = 10240
_NW = 160
_C = 2560
_NCHUNK = _E // _C
_B = 128
_CAP = _C + _B
_BIG = 3.0e38


def _node_body(x_ref, w_ref, o1, o2, o3, o4):
    y = jnp.dot(x_ref[...], w_ref[...], preferred_element_type=jnp.float32)
    o1[...] = y[:, :_D]
    o2[...] = y[:, _D:2 * _D]
    o3[...] = y[:, 2 * _D:3 * _D]
    o4[...] = y[:, 3 * _D:]


def _node_tables(x_pad, wnode):
    blk = 1280
    g = _NPAD // blk
    out = jax.ShapeDtypeStruct((_NPAD, _D), jnp.float32)
    return pl.pallas_call(
        _node_body,
        grid=(g,),
        in_specs=[
            pl.BlockSpec((blk, _D), lambda i: (i, 0)),
            pl.BlockSpec((_D, 4 * _D), lambda i: (0, 0)),
        ],
        out_specs=[pl.BlockSpec((blk, _D), lambda i: (i, 0))] * 4,
        out_shape=[out] * 4,
    )(x_pad, wnode)


def _edge_body(ea_ref, w_ref, b_ref, o1, o2):
    y = (jnp.dot(ea_ref[...], w_ref[...], preferred_element_type=jnp.float32)
         + b_ref[...])
    o1[...] = y[:, :_D]
    o2[...] = y[:, _D:]


def _edge_tables(edge_attr, wea, bias):
    blk = 4000
    g = _E // blk
    ed = edge_attr.shape[1]
    out = jax.ShapeDtypeStruct((_E, _D), jnp.float32)
    return pl.pallas_call(
        _edge_body,
        grid=(g,),
        in_specs=[
            pl.BlockSpec((blk, ed), lambda i: (i, 0)),
            pl.BlockSpec((ed, 2 * _D), lambda i: (0, 0)),
            pl.BlockSpec((2 * _D,), lambda i: (0,)),
        ],
        out_specs=[pl.BlockSpec((blk, _D), lambda i: (i, 0))] * 2,
        out_shape=[out] * 2,
    )(edge_attr, wea, bias)


def _sc_body(dst_h, src_h, me_h, as_h,
             sum_h, sq_h, mn_h, mx_h, cnt_h,
             accS, accQ, accMn, accMx, cntT,
             dstc, srcc, ceid, csrc, cdst, gAs, gme, eIdx, sIdx, sem1, sem2):
    cid = lax.axis_index("c")
    sid = lax.axis_index("s")
    wid = sid * 2 + cid
    zeros_f = jnp.zeros((16,), jnp.float32)
    zeros_i = jnp.zeros((16,), jnp.int32)
    ones_f = jnp.ones((16,), jnp.float32)
    big_f = jnp.full((16,), _BIG, jnp.float32)
    iota16 = lax.iota(jnp.int32, 16)

    def zb(i, c):
        s = pl.ds(i * 16, 16)
        ceid[s] = zeros_i
        csrc[s] = zeros_i
        cdst[s] = zeros_i
        return c
    lax.fori_loop(0, _CAP // 16, zb, 0)

    def process_batch(off, n, full):
        for t in range(_B // 16):
            sl = pl.ds(off + t * 16, 16)
            eIdx[pl.ds(t * 16, 16)] = ceid[sl]
            sIdx[pl.ds(t * 16, 16)] = csrc[sl]
        c1 = pltpu.async_copy(me_h.at[eIdx], gme, sem1)
        c2 = pltpu.async_copy(as_h.at[sIdx], gAs, sem2)
        c1.wait()
        c2.wait()

        def vgroup(v, c):
            base = v * 16
            dlocv = cdst[pl.ds(off + base, 16)]
            for l in range(16):
                j = base + l
                dloc = dlocv[l]

                def accum():
                    for q in range(8):
                        s = pl.ds(q * 16, 16)
                        mq = gAs[j, s] + gme[j, s]
                        plsc.addupdate(accS.at[dloc, s], mq)
                        plsc.addupdate(accQ.at[dloc, s], mq * mq)
                        accMn[dloc, s] = jnp.minimum(accMn[dloc, s], mq)
                        accMx[dloc, s] = jnp.maximum(accMx[dloc, s], mq)

                if full:
                    accum()
                else:
                    pl.when(j < n)(accum)
            return c
        lax.fori_loop(0, _B // 16, vgroup, 0)

    for w in range(2):
        lo = wid * 320 + w * _NW

        def init_row(i, c):
            for q in range(8):
                s = pl.ds(q * 16, 16)
                accS[i, s] = zeros_f
                accQ[i, s] = zeros_f
                accMn[i, s] = big_f
                accMx[i, s] = -big_f
            return c
        lax.fori_loop(0, _NW, init_row, 0)

        def init_cnt(i, c):
            cntT[pl.ds(i * 16, 16)] = zeros_f
            return c
        lax.fori_loop(0, _NW // 16, init_cnt, 0)

        def chunk_body(ci, cursor):
            cb = ci * _C
            pltpu.sync_copy(dst_h.at[pl.ds(cb, _C)], dstc)
            pltpu.sync_copy(src_h.at[pl.ds(cb, _C)], srcc)

            def vbody(v, cur):
                s = pl.ds(v * 16, 16)
                dv = dstc[s]
                sv = srcc[s]
                dloc = dv - lo
                msk = (dloc >= 0) & (dloc < _NW)
                ev = cb + v * 16 + iota16
                plsc.store_compressed(ceid.at[pl.ds(cur, 16)], ev, mask=msk)
                plsc.store_compressed(csrc.at[pl.ds(cur, 16)], sv, mask=msk)
                plsc.store_compressed(cdst.at[pl.ds(cur, 16)], dloc, mask=msk)
                plsc.addupdate_scatter(cntT, [dloc], ones_f, mask=msk)
                return cur + jnp.sum(msk.astype(jnp.int32))
            cursor = lax.fori_loop(0, _C // 16, vbody, cursor)

            nb = cursor >> 7

            def bbody(b, c):
                process_batch(b * _B, _B, True)
                return c
            lax.fori_loop(0, nb, bbody, 0)

            off = nb * _B
            for t in range(_B // 16):
                sl_d = pl.ds(t * 16, 16)
                sl_s = pl.ds(off + t * 16, 16)
                ev = ceid[sl_s]
                sv = csrc[sl_s]
                dv = cdst[sl_s]
                ceid[sl_d] = ev
                csrc[sl_d] = sv
                cdst[sl_d] = dv
            return cursor - off

        cursor = lax.fori_loop(0, _NCHUNK, chunk_body, jnp.int32(0))
        process_batch(0, cursor, False)

        pltpu.sync_copy(accS, sum_h.at[pl.ds(lo, _NW)])
        pltpu.sync_copy(accQ, sq_h.at[pl.ds(lo, _NW)])
        pltpu.sync_copy(accMn, mn_h.at[pl.ds(lo, _NW)])
        pltpu.sync_copy(accMx, mx_h.at[pl.ds(lo, _NW)])
        pltpu.sync_copy(cntT, cnt_h.at[pl.ds(lo, _NW)])


def _sc_stats(dst, src, me, as_):
    mesh = plsc.VectorSubcoreMesh(core_axis_name="c", subcore_axis_name="s",
                                  num_cores=2, num_subcores=16)
    mat = jax.ShapeDtypeStruct((_NPAD, _D), jnp.float32)
    vec = jax.ShapeDtypeStruct((_NPAD,), jnp.float32)
    f = pl.kernel(
        _sc_body,
        out_type=[mat, mat, mat, mat, vec],
        mesh=mesh,
        compiler_params=pltpu.CompilerParams(needs_layout_passes=False),
        scratch_types=[
            pltpu.VMEM((_NW, _D), jnp.float32),
            pltpu.VMEM((_NW, _D), jnp.float32),
            pltpu.VMEM((_NW, _D), jnp.float32),
            pltpu.VMEM((_NW, _D), jnp.float32),
            pltpu.VMEM((_NW,), jnp.float32),
            pltpu.VMEM((_C,), jnp.int32),
            pltpu.VMEM((_C,), jnp.int32),
            pltpu.VMEM((_CAP,), jnp.int32),
            pltpu.VMEM((_CAP,), jnp.int32),
            pltpu.VMEM((_CAP,), jnp.int32),
            pltpu.VMEM((_B, _D), jnp.float32),
            pltpu.VMEM((_B, _D), jnp.float32),
            pltpu.VMEM((_B,), jnp.int32),
            pltpu.VMEM((_B,), jnp.int32),
            pltpu.SemaphoreType.DMA,
            pltpu.SemaphoreType.DMA,
        ],
    )
    return f(dst, src, me, as_)


def _post_body(x_ref, adf, sf, qf, nf, xf, cf, adb, sb, qb, nb, xb, cb,
               wpf, bpf, wlf, blf, wpb, bpb, wlb, blb, whl, bhl, o_ref):
    x = x_ref[...]

    def dir_out(ad_ref, s, q, mn, mx, c, wpost, bpost, wlin, blin):
        ad = ad_ref[...]
        cnt = c[...]
        denom = jnp.maximum(cnt, 1.0)
        has = cnt > 0.0
        adv = jnp.where(has, ad, 0.0)
        meanG = s[...] / denom
        mean = meanG + adv
        mean2 = q[...] / denom + 2.0 * adv * meanG + adv * adv
        var = mean2 - mean * mean
        std = jnp.where(var <= 1e-5, 0.0,
                        jnp.sqrt(jnp.maximum(var, 1e-5)))
        mnv = jnp.where(has, mn[...] + ad, 0.0)
        mxv = jnp.where(has, mx[...] + ad, 0.0)
        agg = jnp.concatenate([mean, mnv, mxv, std], axis=-1)
        logd = jnp.log(denom + 1.0)
        h = jnp.concatenate(
            [x, agg, agg * (logd / _AVG_LOG), agg * (_AVG_LOG / logd)],
            axis=-1)
        p = (jnp.dot(h, wpost[...], preferred_element_type=jnp.float32)
             + bpost[...])
        return (jnp.dot(p, wlin[...], preferred_element_type=jnp.float32)
                + blin[...])

    a_in = dir_out(adf, sf, qf, nf, xf, cf, wpf, bpf, wlf, blf)
    a_out = dir_out(adb, sb, qb, nb, xb, cb, wpb, bpb, wlb, blb)
    hcat = jnp.concatenate([x, a_in, a_out], axis=-1)
    o_ref[...] = (jnp.dot(hcat, whl[...], preferred_element_type=jnp.float32)
                  + bhl[...])


def _post(x, stats_f, stats_b, wpf, bpf, wlf, blf, wpb, bpb, wlb, blb,
          whl, bhl):
    blk = 1000
    g = _N // blk
    m_spec = pl.BlockSpec((blk, _D), lambda i: (i, 0))
    c_spec = pl.BlockSpec((blk, 1), lambda i: (i, 0))
    w13 = pl.BlockSpec((13 * _D, _D), lambda i: (0, 0))
    wdd = pl.BlockSpec((_D, _D), lambda i: (0, 0))
    w3 = pl.BlockSpec((3 * _D, _D), lambda i: (0, 0))
    b_spec = pl.BlockSpec((_D,), lambda i: (0,))
    return pl.pallas_call(
        _post_body,
        grid=(g,),
        in_specs=[m_spec,
                  m_spec, m_spec, m_spec, m_spec, m_spec, c_spec,
                  m_spec, m_spec, m_spec, m_spec, m_spec, c_spec,
                  w13, b_spec, wdd, b_spec,
                  w13, b_spec, wdd, b_spec,
                  w3, b_spec],
        out_specs=pl.BlockSpec((blk, _D), lambda i: (i, 0)),
        out_shape=jax.ShapeDtypeStruct((_N, _D), jnp.float32),
    )(x, *stats_f, *stats_b, wpf, bpf, wlf, blf, wpb, bpb, wlb, blb,
      whl, bhl)


def kernel(x, edge_index, edge_attr, We_f, be_f, Wpre_f, bpre_f, Wpost_f,
           bpost_f, Wlin_f, blin_f, We_b, be_b, Wpre_b, bpre_b, Wpost_b,
           bpost_b, Wlin_b, blin_b, Whl, bhl):
    src = edge_index[0]
    dst = edge_index[1]
    x_pad = jnp.pad(x, ((0, _NPAD - _N), (0, 0)))
    wnode = jnp.concatenate(
        [Wpre_f[:_D], Wpre_f[_D:2 * _D], Wpre_b[:_D], Wpre_b[_D:2 * _D]],
        axis=1)
    ad_f, as_f, ad_b, as_b = _node_tables(x_pad, wnode)
    wea = jnp.concatenate(
        [We_f @ Wpre_f[2 * _D:], We_b @ Wpre_b[2 * _D:]], axis=1)
    bias_e = jnp.concatenate(
        [be_f @ Wpre_f[2 * _D:] + bpre_f, be_b @ Wpre_b[2 * _D:] + bpre_b])
    me_f, me_b = _edge_tables(edge_attr, wea, bias_e)

    s_f, q_f, n_f, x_f, c_f = _sc_stats(dst, src, me_f, as_f)
    s_b, q_b, n_b, x_b, c_b = _sc_stats(src, dst, me_b, as_b)

    stats_f = (ad_f[:_N], s_f[:_N], q_f[:_N], n_f[:_N], x_f[:_N],
               c_f[:_N].reshape(_N, 1))
    stats_b = (ad_b[:_N], s_b[:_N], q_b[:_N], n_b[:_N], x_b[:_N],
               c_b[:_N].reshape(_N, 1))
    return _post(x, stats_f, stats_b, Wpost_f, bpost_f, Wlin_f, blin_f,
                 Wpost_b, bpost_b, Wlin_b, blin_b, Whl, bhl)
